# baseline (speedup 1.0000x reference)
import functools

import jax
import jax.numpy as jnp
from jax import lax
from jax.experimental import pallas as pl
from jax.experimental.pallas import tpu as pltpu
from jax.experimental.pallas import tpu_sc as plsc

B, S, D = 4, 2048, 1024
NC, NS, L = 2, 16, 16
NW = NC * NS
S_PER_W = S // NW
CH = 16


def _tc_add_kernel(x_ref, t_ref, o_ref):
    o_ref[...] = x_ref[...] + t_ref[...][None, :, :]


def _kernel_tc(x, table):
    TS = 512
    return pl.pallas_call(
        _tc_add_kernel,
        grid=(S // TS,),
        in_specs=[
            pl.BlockSpec((B, TS, D), lambda s: (0, s, 0)),
            pl.BlockSpec((TS, D), lambda s: (s, 0)),
        ],
        out_specs=pl.BlockSpec((B, TS, D), lambda s: (0, s, 0)),
        out_shape=jax.ShapeDtypeStruct((B, S, D), x.dtype),
    )(x, table[:S])


_sc_mesh = plsc.VectorSubcoreMesh(core_axis_name="c", subcore_axis_name="s")


@functools.partial(
    pl.kernel,
    mesh=_sc_mesh,
    out_type=jax.ShapeDtypeStruct((B, S, D), jnp.float32),
    scratch_types=[
        pltpu.VMEM((S_PER_W, D), jnp.float32),
        pltpu.VMEM((CH, D), jnp.float32),
    ],
)
def _sc_body(x_hbm, t_hbm, out_hbm, tbuf, xbuf):
    wid = lax.axis_index("s") * NC + lax.axis_index("c")
    base = wid * S_PER_W
    pltpu.sync_copy(t_hbm.at[pl.ds(base, S_PER_W)], tbuf)
    for c in range(S_PER_W // CH):
        for b in range(B):
            s0 = base + c * CH
            pltpu.sync_copy(x_hbm.at[b, pl.ds(s0, CH)], xbuf)

            def row_add(i, _):
                for j in range(D // L):
                    sl = pl.ds(j * L, L)
                    xbuf[i, sl] = xbuf[i, sl] + tbuf[c * CH + i, sl]
                return 0

            lax.fori_loop(0, CH, row_add, 0)
            pltpu.sync_copy(xbuf, out_hbm.at[b, pl.ds(s0, CH)])


def _kernel_sc(x, table):
    return _sc_body(x, table[:S])


kernel = _kernel_sc
